# baseline (speedup 1.0000x reference)
import functools

import jax
import jax.numpy as jnp
from jax import lax
from jax.experimental import pallas as pl
from jax.experimental.pallas import tpu as pltpu
from jax.experimental.pallas import tpu_sc as plsc

NUM_CORES = 2
NUM_SUBCORES = 16
NUM_WORKERS = NUM_CORES * NUM_SUBCORES

B_TC = 8192
T = 512


def _tc_part(x_tc, embed_table):
    V, D = embed_table.shape
    VP = 128
    table_pad = jnp.zeros((VP, D), embed_table.dtype).at[:V].set(embed_table)
    nb = B_TC // T
    x3 = x_tc.reshape(nb, 1, T)

    def body(x_ref, tab_ref, o_ref):
        xv = x_ref[0, 0, :]
        oh = (xv[:, None] == lax.broadcasted_iota(jnp.int32, (T, VP), 1)
              ).astype(jnp.float32)
        o_ref[...] = jnp.dot(oh, tab_ref[...],
                             preferred_element_type=jnp.float32)

    return pl.pallas_call(
        body,
        grid=(nb,),
        in_specs=[
            pl.BlockSpec((1, 1, T), lambda i: (i, 0, 0)),
            pl.BlockSpec((VP, D), lambda i: (0, 0)),
        ],
        out_specs=pl.BlockSpec((T, D), lambda i: (i, 0)),
        out_shape=jax.ShapeDtypeStruct((B_TC, D), jnp.float32),
    )(x3, table_pad)


def _sc_part(x_sc, embed_table):
    (Bs,) = x_sc.shape
    V, D = embed_table.shape
    b_per_w = Bs // NUM_WORKERS
    K = 4
    n_groups = b_per_w // K

    mesh = plsc.VectorSubcoreMesh(core_axis_name="c", subcore_axis_name="s")

    @functools.partial(
        pl.kernel,
        mesh=mesh,
        out_type=jax.ShapeDtypeStruct((Bs, D), jnp.float32),
        scratch_types=[
            pltpu.SMEM((b_per_w,), jnp.int32),
            pltpu.VMEM_SHARED((NUM_WORKERS, b_per_w), jnp.int32),
            pltpu.VMEM((V, D), jnp.float32),
            pltpu.SemaphoreType.DMA,
        ],
    )
    def sc_lookup(table_hbm, idx_hbm, out_hbm, idx_s, idx_v, table_v, sem):
        wid = lax.axis_index("s") * NUM_CORES + lax.axis_index("c")
        base = wid * b_per_w
        pltpu.sync_copy(table_hbm, table_v)
        pltpu.sync_copy(idx_hbm.at[pl.ds(base, b_per_w)], idx_v.at[wid])
        pltpu.sync_copy(idx_v.at[wid], idx_s)

        def fire(r):
            pltpu.async_copy(table_v.at[idx_s[r]], out_hbm.at[base + r], sem)

        def drain_one():
            pltpu.make_async_copy(
                table_hbm.at[0], out_hbm.at[base], sem
            ).wait()

        for j in range(K):
            fire(j)

        def body(g, carry):
            for j in range(K):
                fire(g * K + j)
            for j in range(K):
                drain_one()
            return carry

        lax.fori_loop(1, n_groups, body, 0)
        for j in range(K):
            drain_one()

    return sc_lookup(embed_table, x_sc)


def kernel(x, embed_table):
    x = x.astype(jnp.int32)
    (B,) = x.shape
    sc_out = _sc_part(lax.slice(x, [B_TC], [B]), embed_table)
    tc_out = _tc_part(lax.slice(x, [0], [B_TC]), embed_table)
    return jnp.concatenate([tc_out, sc_out], axis=0)


# speedup vs baseline: 2.1112x; 2.1112x over previous
import functools

import jax
import jax.numpy as jnp
from jax import lax
from jax.experimental import pallas as pl
from jax.experimental.pallas import tpu as pltpu
from jax.experimental.pallas import tpu_sc as plsc

NUM_CORES = 2
NUM_SUBCORES = 16
NUM_WORKERS = NUM_CORES * NUM_SUBCORES

B_TC = 12288
T = 512


def _sc_part(x, embed_table):
    (B,) = x.shape
    V, D = embed_table.shape
    b_per_w = (B - B_TC) // NUM_WORKERS
    K = 4
    n_groups = b_per_w // K

    mesh = plsc.VectorSubcoreMesh(core_axis_name="c", subcore_axis_name="s")

    @functools.partial(
        pl.kernel,
        mesh=mesh,
        out_type=jax.ShapeDtypeStruct((B, D), jnp.float32),
        scratch_types=[
            pltpu.SMEM((b_per_w,), jnp.int32),
            pltpu.VMEM_SHARED((NUM_WORKERS, b_per_w), jnp.int32),
            pltpu.VMEM((V, D), jnp.float32),
            pltpu.SemaphoreType.DMA,
        ],
    )
    def sc_lookup(table_hbm, idx_hbm, out_hbm, idx_s, idx_v, table_v, sem):
        wid = lax.axis_index("s") * NUM_CORES + lax.axis_index("c")
        base = B_TC + wid * b_per_w
        pltpu.sync_copy(table_hbm, table_v)
        pltpu.sync_copy(idx_hbm.at[pl.ds(base, b_per_w)], idx_v.at[wid])
        pltpu.sync_copy(idx_v.at[wid], idx_s)

        def fire(r):
            pltpu.async_copy(table_v.at[idx_s[r]], out_hbm.at[base + r], sem)

        def drain_one():
            pltpu.make_async_copy(
                table_hbm.at[0], out_hbm.at[base], sem
            ).wait()

        for j in range(K):
            fire(j)

        def body(g, carry):
            for j in range(K):
                fire(g * K + j)
            for j in range(K):
                drain_one()
            return carry

        lax.fori_loop(1, n_groups, body, 0)
        for j in range(K):
            drain_one()

    return sc_lookup(embed_table, x)


def _tc_fill(x, embed_table, buf):
    (B,) = x.shape
    V, D = embed_table.shape
    VP = 128
    table_pad = jnp.zeros((VP, D), embed_table.dtype).at[:V].set(embed_table)
    nb = B_TC // T
    x3 = lax.slice(x, [0], [B_TC]).reshape(nb, 1, T)

    def body(x_ref, tab_ref, buf_ref, o_ref):
        xv = x_ref[0, 0, :]
        oh = (xv[:, None] == lax.broadcasted_iota(jnp.int32, (T, VP), 1)
              ).astype(jnp.float32)
        o_ref[...] = jnp.dot(oh, tab_ref[...],
                             preferred_element_type=jnp.float32)

    return pl.pallas_call(
        body,
        grid=(nb,),
        in_specs=[
            pl.BlockSpec((1, 1, T), lambda i: (i, 0, 0)),
            pl.BlockSpec((VP, D), lambda i: (0, 0)),
            pl.BlockSpec(memory_space=pl.ANY),
        ],
        out_specs=pl.BlockSpec((T, D), lambda i: (i, 0)),
        out_shape=jax.ShapeDtypeStruct((B, D), jnp.float32),
        input_output_aliases={2: 0},
    )(x3, table_pad, buf)


def kernel(x, embed_table):
    x = x.astype(jnp.int32)
    sc_full = _sc_part(x, embed_table)
    return _tc_fill(x, embed_table, sc_full)


# speedup vs baseline: 2.2523x; 1.0668x over previous
import functools

import jax
import jax.numpy as jnp
from jax import lax
from jax.experimental import pallas as pl
from jax.experimental.pallas import tpu as pltpu
from jax.experimental.pallas import tpu_sc as plsc

NUM_CORES = 2
NUM_SUBCORES = 16
NUM_WORKERS = NUM_CORES * NUM_SUBCORES


def kernel(x, embed_table):
    x = x.astype(jnp.int32)
    (B,) = x.shape
    V, D = embed_table.shape
    b_per_w = B // NUM_WORKERS
    K = 4
    n_groups = b_per_w // K

    mesh = plsc.VectorSubcoreMesh(core_axis_name="c", subcore_axis_name="s")

    @functools.partial(
        pl.kernel,
        mesh=mesh,
        out_type=jax.ShapeDtypeStruct((B, D), jnp.float32),
        scratch_types=[
            pltpu.SMEM((b_per_w,), jnp.int32),
            pltpu.VMEM_SHARED((NUM_WORKERS, b_per_w), jnp.int32),
            pltpu.VMEM((V, D), jnp.float32),
            pltpu.SemaphoreType.DMA,
        ],
    )
    def sc_lookup(table_hbm, idx_hbm, out_hbm, idx_s, idx_v, table_v, sem):
        wid = lax.axis_index("s") * NUM_CORES + lax.axis_index("c")
        base = wid * b_per_w
        pltpu.sync_copy(table_hbm, table_v)
        pltpu.sync_copy(idx_hbm.at[pl.ds(base, b_per_w)], idx_v.at[wid])
        pltpu.sync_copy(idx_v.at[wid], idx_s)

        def fire(r):
            pltpu.async_copy(table_v.at[idx_s[r]], out_hbm.at[base + r], sem)

        def drain_one():
            pltpu.make_async_copy(
                table_hbm.at[0], out_hbm.at[base], sem
            ).wait()

        for j in range(K):
            fire(j)

        def body(g, carry):
            for j in range(K):
                fire(g * K + j)
            for j in range(K):
                drain_one()
            return carry

        lax.fori_loop(1, n_groups, body, 0)
        for j in range(K):
            drain_one()

    return sc_lookup(embed_table, x)
